# Initial kernel scaffold; baseline (speedup 1.0000x reference)
import jax
import jax.numpy as jnp
from jax.experimental import pallas as pl


def kernel(l_feat, r_feat, W0, asrc0, adst0, W1, asrc1, adst1, W2, asrc2, adst2, W3, asrc3, adst3):
    raise NotImplementedError("write your pallas kernel here")


# speedup vs baseline: 3.1228x; 3.1228x over previous
import functools

import jax
import jax.numpy as jnp
from jax.experimental import pallas as pl
from jax.experimental.pallas import tpu as pltpu

HEADS = 4
HEAD = 64
THRESH = 0.15
LAYERS = 4
NEG = -1e30


def _leaky(x):
    return jnp.where(x >= 0, x, 0.2 * x)


def _elu(x):
    return jnp.where(x > 0, x, jnp.exp(jnp.minimum(x, 0.0)) - 1.0)


def _mask_kernel(xr_ref, xl_ref, out_ref):
    xr = xr_ref[...]
    xl = xl_ref[...]
    rn = xr / (jnp.sqrt(jnp.sum(xr * xr, axis=1, keepdims=True)) + 1e-12)
    ln = xl / (jnp.sqrt(jnp.sum(xl * xl, axis=1, keepdims=True)) + 1e-12)
    sim = jax.lax.dot_general(
        rn, ln, (((1,), (1,)), ((), ())),
        precision=jax.lax.Precision.HIGHEST,
        preferred_element_type=jnp.float32)
    out_ref[...] = (sim > THRESH).astype(jnp.int8)


def _proj_kernel(h_ref, w_ref, wh_ref, pre_ref, *, act):
    wh = jnp.dot(h_ref[...], w_ref[...], preferred_element_type=jnp.float32)
    wh_ref[...] = wh
    pre_ref[...] = _elu(wh) if act else wh


def _attn_kernel(whl_ref, whr_ref, mask_ref, asrc_ref, adst_ref, out_ref,
                 acc_ref, m_ref, den_ref, *, act):
    j = pl.program_id(1)
    nj = pl.num_programs(1)

    @pl.when(j == 0)
    def _init():
        acc_ref[...] = jnp.zeros_like(acc_ref)
        m_ref[...] = jnp.full_like(m_ref, NEG)
        den_ref[...] = jnp.zeros_like(den_ref)

    whl = whl_ref[...]
    whr = whr_ref[...]
    mask = mask_ref[...] != 0

    for h in range(HEADS):
        sl = slice(h * HEAD, (h + 1) * HEAD)
        whl_h = whl[:, sl]
        es_l = jnp.sum(whl_h * asrc_ref[h, :][None, :], axis=1)
        ed_r = jnp.sum(whr[:, sl] * adst_ref[h, :][None, :], axis=1)
        e = _leaky(ed_r[:, None] + es_l[None, :])
        e = jnp.where(mask, e, NEG)
        m_old = m_ref[:, h:h + 1]
        m_new = jnp.maximum(m_old, jnp.max(e, axis=1, keepdims=True))
        scale = jnp.exp(m_old - m_new)
        ex = jnp.where(mask, jnp.exp(e - m_new), 0.0)
        m_ref[:, h:h + 1] = m_new
        den_ref[:, h:h + 1] = den_ref[:, h:h + 1] * scale + jnp.sum(
            ex, axis=1, keepdims=True)
        acc_ref[:, sl] = acc_ref[:, sl] * scale + jnp.dot(
            ex, whl_h, preferred_element_type=jnp.float32)

    @pl.when(j == nj - 1)
    def _finish():
        outs = []
        for h in range(HEADS):
            sl = slice(h * HEAD, (h + 1) * HEAD)
            whr_h = whr[:, sl]
            es_r = jnp.sum(whr_h * asrc_ref[h, :][None, :], axis=1,
                           keepdims=True)
            ed_r = jnp.sum(whr_h * adst_ref[h, :][None, :], axis=1,
                           keepdims=True)
            e_self = _leaky(es_r + ed_r)
            m_old = m_ref[:, h:h + 1]
            m_fin = jnp.maximum(m_old, e_self)
            scale = jnp.exp(m_old - m_fin)
            ex_self = jnp.exp(e_self - m_fin)
            den = den_ref[:, h:h + 1] * scale + ex_self
            num = acc_ref[:, sl] * scale + ex_self * whr_h
            outs.append(num / (den + 1e-9))
        o = jnp.concatenate(outs, axis=1)
        out_ref[...] = _elu(o) if act else o


def _gather_kernel(h_ref, w_ref, asrc_ref, adst_ref, out_ref):
    hb = h_ref[...]
    P, Dh = hb.shape
    for i in range(LAYERS):
        wh = jnp.dot(hb, w_ref[i], preferred_element_type=jnp.float32)
        parts = []
        for h in range(HEADS):
            sl = slice(h * HEAD, (h + 1) * HEAD)
            wh_h = wh[:, sl]
            es = jnp.sum(wh_h * asrc_ref[i, h][None, :], axis=1,
                         keepdims=True)
            ed0 = jnp.sum(wh_h[0:1, :] * adst_ref[i, h][None, :], axis=1,
                          keepdims=True)
            e = _leaky(es + ed0)
            m = jnp.max(e, axis=0, keepdims=True)
            ex = jnp.exp(e - m)
            den = jnp.sum(ex, axis=0, keepdims=True) + ex[0:1, :]
            num = jnp.sum(ex * wh_h, axis=0, keepdims=True) \
                + ex[0:1, :] * wh_h[0:1, :]
            parts.append(num / (den + 1e-9))
        row0 = jnp.concatenate(parts, axis=1)
        if i < LAYERS - 1:
            body = _elu(wh)
            row0 = _elu(row0)
            rid = jax.lax.broadcasted_iota(jnp.int32, (P, Dh), 0)
            hb = jnp.where(rid == 0, jnp.broadcast_to(row0, (P, Dh)), body)
        else:
            out_ref[...] = row0[None, :, :]


def kernel(l_feat, r_feat, W0, asrc0, adst0, W1, asrc1, adst1,
           W2, asrc2, adst2, W3, asrc3, adst3):
    Ws = [W0, W1, W2, W3]
    As = [asrc0, asrc1, asrc2, asrc3]
    Ad = [adst0, adst1, adst2, adst3]
    B, P, Dh = l_feat.shape
    node_l = l_feat.reshape(-1, Dh)
    node_r = r_feat.reshape(-1, Dh)
    L = node_l.shape[0]
    R = node_r.shape[0]
    N = L + R

    TR = min(256, R)
    TL = min(256, L)
    TM = min(256, N)

    mask = pl.pallas_call(
        _mask_kernel,
        grid=(R // TR, L // TL),
        in_specs=[
            pl.BlockSpec((TR, Dh), lambda i, j: (i, 0)),
            pl.BlockSpec((TL, Dh), lambda i, j: (j, 0)),
        ],
        out_specs=pl.BlockSpec((TR, TL), lambda i, j: (i, j)),
        out_shape=jax.ShapeDtypeStruct((R, L), jnp.int8),
        compiler_params=pltpu.CompilerParams(
            dimension_semantics=("parallel", "parallel")),
    )(node_r, node_l)

    h = jnp.concatenate([node_l, node_r], axis=0)
    for i in range(LAYERS):
        act = i < LAYERS - 1
        wh, pre = pl.pallas_call(
            functools.partial(_proj_kernel, act=act),
            grid=(N // TM,),
            in_specs=[
                pl.BlockSpec((TM, Dh), lambda t: (t, 0)),
                pl.BlockSpec((Dh, Dh), lambda t: (0, 0)),
            ],
            out_specs=[
                pl.BlockSpec((TM, Dh), lambda t: (t, 0)),
                pl.BlockSpec((TM, Dh), lambda t: (t, 0)),
            ],
            out_shape=[
                jax.ShapeDtypeStruct((N, Dh), jnp.float32),
                jax.ShapeDtypeStruct((N, Dh), jnp.float32),
            ],
            compiler_params=pltpu.CompilerParams(
                dimension_semantics=("parallel",)),
        )(h, Ws[i])

        lofs = L // TR
        new_r = pl.pallas_call(
            functools.partial(_attn_kernel, act=act),
            grid=(R // TR, L // TL),
            in_specs=[
                pl.BlockSpec((TL, Dh), lambda i, j: (j, 0)),
                pl.BlockSpec((TR, Dh), lambda i, j, o=lofs: (i + o, 0)),
                pl.BlockSpec((TR, TL), lambda i, j: (i, j)),
                pl.BlockSpec((HEADS, HEAD), lambda i, j: (0, 0)),
                pl.BlockSpec((HEADS, HEAD), lambda i, j: (0, 0)),
            ],
            out_specs=pl.BlockSpec((TR, Dh), lambda i, j: (i, 0)),
            out_shape=jax.ShapeDtypeStruct((R, Dh), jnp.float32),
            scratch_shapes=[
                pltpu.VMEM((TR, Dh), jnp.float32),
                pltpu.VMEM((TR, 128), jnp.float32),
                pltpu.VMEM((TR, 128), jnp.float32),
            ],
            compiler_params=pltpu.CompilerParams(
                dimension_semantics=("parallel", "arbitrary")),
        )(wh, wh, mask, As[i], Ad[i])

        h = jnp.concatenate([pre[:L], new_r], axis=0)

    NB = N // P
    wstack = jnp.stack(Ws)
    astack = jnp.stack(As)
    adstack = jnp.stack(Ad)
    g = pl.pallas_call(
        _gather_kernel,
        grid=(NB,),
        in_specs=[
            pl.BlockSpec((P, Dh), lambda b: (b, 0)),
            pl.BlockSpec((LAYERS, Dh, Dh), lambda b: (0, 0, 0)),
            pl.BlockSpec((LAYERS, HEADS, HEAD), lambda b: (0, 0, 0)),
            pl.BlockSpec((LAYERS, HEADS, HEAD), lambda b: (0, 0, 0)),
        ],
        out_specs=pl.BlockSpec((1, 1, Dh), lambda b: (b, 0, 0)),
        out_shape=jax.ShapeDtypeStruct((NB, 1, Dh), jnp.float32),
        compiler_params=pltpu.CompilerParams(
            dimension_semantics=("parallel",)),
    )(h, wstack, astack, adstack)

    g = g.reshape(NB, Dh)
    return g[:B], g[B:]


# speedup vs baseline: 5.1875x; 1.6612x over previous
import functools

import jax
import jax.numpy as jnp
from jax.experimental import pallas as pl
from jax.experimental.pallas import tpu as pltpu

HEADS = 4
HEAD = 64
THRESH = 0.15
LAYERS = 4
NEG = -1e30


def _leaky(x):
    return jnp.maximum(x, 0.2 * x)


def _elu(x):
    return jnp.where(x > 0, x, jnp.exp(jnp.minimum(x, 0.0)) - 1.0)


def _mask_kernel(xr_ref, xl_ref, out_ref):
    xr = xr_ref[...]
    xl = xl_ref[...]
    rn = xr / (jnp.sqrt(jnp.sum(xr * xr, axis=1, keepdims=True)) + 1e-12)
    ln = xl / (jnp.sqrt(jnp.sum(xl * xl, axis=1, keepdims=True)) + 1e-12)
    sim = jax.lax.dot_general(
        rn, ln, (((1,), (1,)), ((), ())),
        precision=jax.lax.Precision.HIGHEST,
        preferred_element_type=jnp.float32)
    out_ref[...] = (sim > THRESH).astype(jnp.int8)


def _proj_kernel(h_ref, w_ref, abig_ref, wh_ref, pre_ref, est_ref, edc_ref,
                 *, act):
    wh = jnp.dot(h_ref[...], w_ref[...], preferred_element_type=jnp.float32)
    wh_ref[...] = wh
    pre_ref[...] = _elu(wh) if act else wh
    abig = abig_ref[...]
    edc_ref[...] = jnp.dot(wh, abig, precision=jax.lax.Precision.HIGHEST,
                           preferred_element_type=jnp.float32)
    est_ref[...] = jax.lax.dot_general(
        abig, wh, (((0,), (1,)), ((), ())),
        precision=jax.lax.Precision.HIGHEST,
        preferred_element_type=jnp.float32)


def _attn_kernel(whl_ref, whr_ref, mask_ref, est_ref, edc_ref, out_ref,
                 acc_ref, m_ref, den_ref, *, act):
    j = pl.program_id(1)
    nj = pl.num_programs(1)
    edc = edc_ref[...]

    @pl.when(j == 0)
    def _init():
        acc_ref[...] = whr_ref[...]
        den_ref[...] = jnp.ones_like(den_ref)
        for h in range(HEADS):
            e_self = _leaky(edc[:, HEADS + h:HEADS + h + 1]
                            + edc[:, h:h + 1])
            m_ref[:, h:h + 1] = e_self

    whl = whl_ref[...]
    bias = (mask_ref[...].astype(jnp.float32) - 1.0) * (-NEG)

    for h in range(HEADS):
        sl = slice(h * HEAD, (h + 1) * HEAD)
        ed_r = edc[:, h:h + 1]
        es_l = est_ref[HEADS + h:HEADS + h + 1, :]
        e = _leaky(ed_r + es_l) + bias
        m_old = m_ref[:, h:h + 1]
        m_new = jnp.maximum(m_old, jnp.max(e, axis=1, keepdims=True))
        scale = jnp.exp(m_old - m_new)
        ex = jnp.exp(e - m_new)
        m_ref[:, h:h + 1] = m_new
        den_ref[:, h:h + 1] = den_ref[:, h:h + 1] * scale + jnp.sum(
            ex, axis=1, keepdims=True)
        acc_ref[:, sl] = acc_ref[:, sl] * scale + jnp.dot(
            ex, whl[:, sl], preferred_element_type=jnp.float32)

    @pl.when(j == nj - 1)
    def _finish():
        for h in range(HEADS):
            sl = slice(h * HEAD, (h + 1) * HEAD)
            o = acc_ref[:, sl] / (den_ref[:, h:h + 1] + 1e-9)
            out_ref[:, sl] = _elu(o) if act else o


def _gather_kernel(h_ref, w_ref, asrc_ref, adst_ref, out_ref):
    hb = h_ref[...]
    P, Dh = hb.shape
    for i in range(LAYERS):
        wh = jnp.dot(hb, w_ref[i], preferred_element_type=jnp.float32)
        parts = []
        for h in range(HEADS):
            sl = slice(h * HEAD, (h + 1) * HEAD)
            wh_h = wh[:, sl]
            es = jnp.sum(wh_h * asrc_ref[i, h][None, :], axis=1,
                         keepdims=True)
            ed0 = jnp.sum(wh_h[0:1, :] * adst_ref[i, h][None, :], axis=1,
                          keepdims=True)
            e = _leaky(es + ed0)
            m = jnp.max(e, axis=0, keepdims=True)
            ex = jnp.exp(e - m)
            den = jnp.sum(ex, axis=0, keepdims=True) + ex[0:1, :]
            num = jnp.sum(ex * wh_h, axis=0, keepdims=True) \
                + ex[0:1, :] * wh_h[0:1, :]
            parts.append(num / (den + 1e-9))
        row0 = jnp.concatenate(parts, axis=1)
        if i < LAYERS - 1:
            body = _elu(wh)
            row0 = _elu(row0)
            rid = jax.lax.broadcasted_iota(jnp.int32, (P, Dh), 0)
            hb = jnp.where(rid == 0, jnp.broadcast_to(row0, (P, Dh)), body)
        else:
            out_ref[...] = row0[None, :, :]


def kernel(l_feat, r_feat, W0, asrc0, adst0, W1, asrc1, adst1,
           W2, asrc2, adst2, W3, asrc3, adst3):
    Ws = [W0, W1, W2, W3]
    As = [asrc0, asrc1, asrc2, asrc3]
    Ad = [adst0, adst1, adst2, adst3]
    B, P, Dh = l_feat.shape
    node_l = l_feat.reshape(-1, Dh)
    node_r = r_feat.reshape(-1, Dh)
    L = node_l.shape[0]
    R = node_r.shape[0]
    N = L + R

    TR = min(256, R)
    TL = min(512, L)
    TM = min(256, N)

    mask = pl.pallas_call(
        _mask_kernel,
        grid=(R // TR, L // TR),
        in_specs=[
            pl.BlockSpec((TR, Dh), lambda i, j: (i, 0)),
            pl.BlockSpec((TR, Dh), lambda i, j: (j, 0)),
        ],
        out_specs=pl.BlockSpec((TR, TR), lambda i, j: (i, j)),
        out_shape=jax.ShapeDtypeStruct((R, L), jnp.int8),
        compiler_params=pltpu.CompilerParams(
            dimension_semantics=("parallel", "parallel")),
    )(node_r, node_l)

    abigs = []
    for i in range(LAYERS):
        a = jnp.zeros((Dh, 2 * HEADS), jnp.float32)
        for h in range(HEADS):
            a = a.at[h * HEAD:(h + 1) * HEAD, h].set(Ad[i][h])
            a = a.at[h * HEAD:(h + 1) * HEAD, HEADS + h].set(As[i][h])
        abigs.append(a)

    h = jnp.concatenate([node_l, node_r], axis=0)
    for i in range(LAYERS):
        act = i < LAYERS - 1
        wh, pre, est, edc = pl.pallas_call(
            functools.partial(_proj_kernel, act=act),
            grid=(N // TM,),
            in_specs=[
                pl.BlockSpec((TM, Dh), lambda t: (t, 0)),
                pl.BlockSpec((Dh, Dh), lambda t: (0, 0)),
                pl.BlockSpec((Dh, 2 * HEADS), lambda t: (0, 0)),
            ],
            out_specs=[
                pl.BlockSpec((TM, Dh), lambda t: (t, 0)),
                pl.BlockSpec((TM, Dh), lambda t: (t, 0)),
                pl.BlockSpec((2 * HEADS, TM), lambda t: (0, t)),
                pl.BlockSpec((TM, 2 * HEADS), lambda t: (t, 0)),
            ],
            out_shape=[
                jax.ShapeDtypeStruct((N, Dh), jnp.float32),
                jax.ShapeDtypeStruct((N, Dh), jnp.float32),
                jax.ShapeDtypeStruct((2 * HEADS, N), jnp.float32),
                jax.ShapeDtypeStruct((N, 2 * HEADS), jnp.float32),
            ],
            compiler_params=pltpu.CompilerParams(
                dimension_semantics=("parallel",)),
        )(h, Ws[i], abigs[i])

        lofs = L // TR
        new_r = pl.pallas_call(
            functools.partial(_attn_kernel, act=act),
            grid=(R // TR, L // TL),
            in_specs=[
                pl.BlockSpec((TL, Dh), lambda i, j: (j, 0)),
                pl.BlockSpec((TR, Dh), lambda i, j, o=lofs: (i + o, 0)),
                pl.BlockSpec((TR, TL), lambda i, j: (i, j)),
                pl.BlockSpec((2 * HEADS, TL), lambda i, j: (0, j)),
                pl.BlockSpec((TR, 2 * HEADS), lambda i, j, o=lofs: (i + o, 0)),
            ],
            out_specs=pl.BlockSpec((TR, Dh), lambda i, j: (i, 0)),
            out_shape=jax.ShapeDtypeStruct((R, Dh), jnp.float32),
            scratch_shapes=[
                pltpu.VMEM((TR, Dh), jnp.float32),
                pltpu.VMEM((TR, 128), jnp.float32),
                pltpu.VMEM((TR, 128), jnp.float32),
            ],
            compiler_params=pltpu.CompilerParams(
                dimension_semantics=("parallel", "arbitrary")),
        )(wh, wh, mask, est, edc)

        h = jnp.concatenate([pre[:L], new_r], axis=0)

    NB = N // P
    wstack = jnp.stack(Ws)
    astack = jnp.stack(As)
    adstack = jnp.stack(Ad)
    g = pl.pallas_call(
        _gather_kernel,
        grid=(NB,),
        in_specs=[
            pl.BlockSpec((P, Dh), lambda b: (b, 0)),
            pl.BlockSpec((LAYERS, Dh, Dh), lambda b: (0, 0, 0)),
            pl.BlockSpec((LAYERS, HEADS, HEAD), lambda b: (0, 0, 0)),
            pl.BlockSpec((LAYERS, HEADS, HEAD), lambda b: (0, 0, 0)),
        ],
        out_specs=pl.BlockSpec((1, 1, Dh), lambda b: (b, 0, 0)),
        out_shape=jax.ShapeDtypeStruct((NB, 1, Dh), jnp.float32),
        compiler_params=pltpu.CompilerParams(
            dimension_semantics=("parallel",)),
    )(h, wstack, astack, adstack)

    g = g.reshape(NB, Dh)
    return g[:B], g[B:]


# speedup vs baseline: 7.8149x; 1.5065x over previous
import functools

import jax
import jax.numpy as jnp
from jax.experimental import pallas as pl
from jax.experimental.pallas import tpu as pltpu

HEADS = 4
HEAD = 64
THRESH = 0.15
LAYERS = 4
NEG = -1e30


def _leaky(x):
    return jnp.maximum(x, 0.2 * x)


def _elu(x):
    return jnp.where(x > 0, x, jnp.exp(jnp.minimum(x, 0.0)) - 1.0)


def _mask_kernel(xr_ref, xl_ref, out_ref):
    xr = xr_ref[...]
    xl = xl_ref[...]
    rn = xr / (jnp.sqrt(jnp.sum(xr * xr, axis=1, keepdims=True)) + 1e-12)
    ln = xl / (jnp.sqrt(jnp.sum(xl * xl, axis=1, keepdims=True)) + 1e-12)
    sim = jax.lax.dot_general(
        rn, ln, (((1,), (1,)), ((), ())),
        precision=jax.lax.Precision.HIGHEST,
        preferred_element_type=jnp.float32)
    out_ref[...] = (sim > THRESH).astype(jnp.int8)


def _proj_kernel(h_ref, w_ref, abig_ref, wh_ref, pre_ref, est_ref, edc_ref,
                 *, act):
    wh = jnp.dot(h_ref[...], w_ref[...], preferred_element_type=jnp.float32)
    wh_ref[...] = wh
    pre_ref[...] = _elu(wh) if act else wh
    abig = abig_ref[...]
    edc_ref[...] = jnp.dot(wh, abig, precision=jax.lax.Precision.HIGHEST,
                           preferred_element_type=jnp.float32)
    est_ref[...] = jax.lax.dot_general(
        abig, wh, (((0,), (1,)), ((), ())),
        precision=jax.lax.Precision.HIGHEST,
        preferred_element_type=jnp.float32)


def _attn_kernel(whl_ref, whr_ref, mask_ref, est_ref, edc_ref, out_ref,
                 *scratch, act):
    accs = scratch[0:HEADS]
    ms = scratch[HEADS:2 * HEADS]
    ds = scratch[2 * HEADS:3 * HEADS]
    j = pl.program_id(1)
    nj = pl.num_programs(1)
    edc = edc_ref[...]

    @pl.when(j == 0)
    def _init():
        whr = whr_ref[...]
        for h in range(HEADS):
            sl = slice(h * HEAD, (h + 1) * HEAD)
            accs[h][...] = whr[:, sl]
            ds[h][...] = jnp.ones_like(ds[h])
            e_self = _leaky(edc[:, HEADS + h:HEADS + h + 1]
                            + edc[:, h:h + 1])
            ms[h][:, 0:1] = e_self

    whl = whl_ref[...]
    TL = whl.shape[0]
    bias = (mask_ref[...].astype(jnp.float32) - 1.0) * (-NEG)
    ones = jnp.ones((TL, 8), jnp.float32)

    for h in range(HEADS):
        sl = slice(h * HEAD, (h + 1) * HEAD)
        ed_r = edc[:, h:h + 1]
        es_l = est_ref[HEADS + h:HEADS + h + 1, :]
        e = _leaky(ed_r + es_l) + bias
        m_old = ms[h][:, 0:1]
        m_new = jnp.maximum(m_old, jnp.max(e, axis=1, keepdims=True))
        scale = jnp.exp2(m_old - m_new)
        ex = jnp.exp2(e - m_new)
        ms[h][:, 0:1] = m_new
        den_inc = jnp.dot(ex, ones, preferred_element_type=jnp.float32)
        ds[h][...] = ds[h][...] * scale + den_inc
        accs[h][...] = accs[h][...] * scale + jnp.dot(
            ex, whl[:, sl], preferred_element_type=jnp.float32)

    @pl.when(j == nj - 1)
    def _finish():
        for h in range(HEADS):
            sl = slice(h * HEAD, (h + 1) * HEAD)
            o = accs[h][...] / (ds[h][:, 0:1] + 1e-9)
            out_ref[:, sl] = _elu(o) if act else o


def _gather_kernel(h_ref, w_ref, asrc_ref, adst_ref, out_ref):
    hb = h_ref[...]
    P, Dh = hb.shape
    for i in range(LAYERS):
        wh = jnp.dot(hb, w_ref[i], preferred_element_type=jnp.float32)
        parts = []
        for h in range(HEADS):
            sl = slice(h * HEAD, (h + 1) * HEAD)
            wh_h = wh[:, sl]
            es = jnp.sum(wh_h * asrc_ref[i, h][None, :], axis=1,
                         keepdims=True)
            ed0 = jnp.sum(wh_h[0:1, :] * adst_ref[i, h][None, :], axis=1,
                          keepdims=True)
            e = _leaky(es + ed0)
            m = jnp.max(e, axis=0, keepdims=True)
            ex = jnp.exp(e - m)
            den = jnp.sum(ex, axis=0, keepdims=True) + ex[0:1, :]
            num = jnp.sum(ex * wh_h, axis=0, keepdims=True) \
                + ex[0:1, :] * wh_h[0:1, :]
            parts.append(num / (den + 1e-9))
        row0 = jnp.concatenate(parts, axis=1)
        if i < LAYERS - 1:
            body = _elu(wh)
            row0 = _elu(row0)
            rid = jax.lax.broadcasted_iota(jnp.int32, (P, Dh), 0)
            hb = jnp.where(rid == 0, jnp.broadcast_to(row0, (P, Dh)), body)
        else:
            out_ref[...] = row0[None, :, :]


def kernel(l_feat, r_feat, W0, asrc0, adst0, W1, asrc1, adst1,
           W2, asrc2, adst2, W3, asrc3, adst3):
    Ws = [W0, W1, W2, W3]
    As = [asrc0, asrc1, asrc2, asrc3]
    Ad = [adst0, adst1, adst2, adst3]
    B, P, Dh = l_feat.shape
    node_l = l_feat.reshape(-1, Dh)
    node_r = r_feat.reshape(-1, Dh)
    L = node_l.shape[0]
    R = node_r.shape[0]
    N = L + R

    TR = min(256, R)
    TL = min(512, L)
    TM = min(256, N)

    mask = pl.pallas_call(
        _mask_kernel,
        grid=(R // TR, L // TR),
        in_specs=[
            pl.BlockSpec((TR, Dh), lambda i, j: (i, 0)),
            pl.BlockSpec((TR, Dh), lambda i, j: (j, 0)),
        ],
        out_specs=pl.BlockSpec((TR, TR), lambda i, j: (i, j)),
        out_shape=jax.ShapeDtypeStruct((R, L), jnp.int8),
        compiler_params=pltpu.CompilerParams(
            dimension_semantics=("parallel", "parallel")),
    )(node_r, node_l)

    log2e = jnp.float32(1.4426950408889634)
    abigs = []
    for i in range(LAYERS):
        a = jnp.zeros((Dh, 2 * HEADS), jnp.float32)
        for h in range(HEADS):
            a = a.at[h * HEAD:(h + 1) * HEAD, h].set(Ad[i][h] * log2e)
            a = a.at[h * HEAD:(h + 1) * HEAD, HEADS + h].set(As[i][h] * log2e)
        abigs.append(a)

    h = jnp.concatenate([node_l, node_r], axis=0)
    for i in range(LAYERS):
        act = i < LAYERS - 1
        wh, pre, est, edc = pl.pallas_call(
            functools.partial(_proj_kernel, act=act),
            grid=(N // TM,),
            in_specs=[
                pl.BlockSpec((TM, Dh), lambda t: (t, 0)),
                pl.BlockSpec((Dh, Dh), lambda t: (0, 0)),
                pl.BlockSpec((Dh, 2 * HEADS), lambda t: (0, 0)),
            ],
            out_specs=[
                pl.BlockSpec((TM, Dh), lambda t: (t, 0)),
                pl.BlockSpec((TM, Dh), lambda t: (t, 0)),
                pl.BlockSpec((2 * HEADS, TM), lambda t: (0, t)),
                pl.BlockSpec((TM, 2 * HEADS), lambda t: (t, 0)),
            ],
            out_shape=[
                jax.ShapeDtypeStruct((N, Dh), jnp.float32),
                jax.ShapeDtypeStruct((N, Dh), jnp.float32),
                jax.ShapeDtypeStruct((2 * HEADS, N), jnp.float32),
                jax.ShapeDtypeStruct((N, 2 * HEADS), jnp.float32),
            ],
            compiler_params=pltpu.CompilerParams(
                dimension_semantics=("parallel",)),
        )(h, Ws[i], abigs[i])

        lofs = L // TR
        new_r = pl.pallas_call(
            functools.partial(_attn_kernel, act=act),
            grid=(R // TR, L // TL),
            in_specs=[
                pl.BlockSpec((TL, Dh), lambda i, j: (j, 0)),
                pl.BlockSpec((TR, Dh), lambda i, j, o=lofs: (i + o, 0)),
                pl.BlockSpec((TR, TL), lambda i, j: (i, j)),
                pl.BlockSpec((2 * HEADS, TL), lambda i, j: (0, j)),
                pl.BlockSpec((TR, 2 * HEADS), lambda i, j, o=lofs: (i + o, 0)),
            ],
            out_specs=pl.BlockSpec((TR, Dh), lambda i, j: (i, 0)),
            out_shape=jax.ShapeDtypeStruct((R, Dh), jnp.float32),
            scratch_shapes=(
                [pltpu.VMEM((TR, HEAD), jnp.float32) for _ in range(HEADS)]
                + [pltpu.VMEM((TR, 8), jnp.float32) for _ in range(HEADS)]
                + [pltpu.VMEM((TR, 8), jnp.float32) for _ in range(HEADS)]
            ),
            compiler_params=pltpu.CompilerParams(
                dimension_semantics=("parallel", "arbitrary")),
        )(wh, wh, mask, est, edc)

        h = jnp.concatenate([pre[:L], new_r], axis=0)

    NB = N // P
    wstack = jnp.stack(Ws)
    astack = jnp.stack(As)
    adstack = jnp.stack(Ad)
    g = pl.pallas_call(
        _gather_kernel,
        grid=(NB,),
        in_specs=[
            pl.BlockSpec((P, Dh), lambda b: (b, 0)),
            pl.BlockSpec((LAYERS, Dh, Dh), lambda b: (0, 0, 0)),
            pl.BlockSpec((LAYERS, HEADS, HEAD), lambda b: (0, 0, 0)),
            pl.BlockSpec((LAYERS, HEADS, HEAD), lambda b: (0, 0, 0)),
        ],
        out_specs=pl.BlockSpec((1, 1, Dh), lambda b: (b, 0, 0)),
        out_shape=jax.ShapeDtypeStruct((NB, 1, Dh), jnp.float32),
        compiler_params=pltpu.CompilerParams(
            dimension_semantics=("parallel",)),
    )(h, wstack, astack, adstack)

    g = g.reshape(NB, Dh)
    return g[:B], g[B:]


# speedup vs baseline: 8.6150x; 1.1024x over previous
import functools

import jax
import jax.numpy as jnp
from jax.experimental import pallas as pl
from jax.experimental.pallas import tpu as pltpu

HEADS = 4
HEAD = 64
THRESH = 0.15
LAYERS = 4
NEG = -1e30


def _leaky(x):
    return jnp.maximum(x, 0.2 * x)


def _elu(x):
    return jnp.where(x > 0, x, jnp.exp(jnp.minimum(x, 0.0)) - 1.0)


def _norm_kernel(x_ref, o_ref):
    x = x_ref[...]
    o_ref[...] = x / (jnp.sqrt(jnp.sum(x * x, axis=1, keepdims=True)) + 1e-12)


def _mask_kernel(rn_ref, ln_ref, out_ref):
    sim = jax.lax.dot_general(
        rn_ref[...], ln_ref[...], (((1,), (1,)), ((), ())),
        precision=jax.lax.Precision.HIGHEST,
        preferred_element_type=jnp.float32)
    out_ref[...] = (sim > THRESH).astype(jnp.int8)


def _proj_kernel(h_ref, w_ref, abig_ref, wh_ref, pre_ref, est_ref, edc_ref,
                 *, act):
    wh = jnp.dot(h_ref[...], w_ref[...], preferred_element_type=jnp.float32)
    wh_ref[...] = wh
    pre_ref[...] = _elu(wh) if act else wh
    abig = abig_ref[...]
    edc_ref[...] = jnp.dot(wh, abig, precision=jax.lax.Precision.HIGHEST,
                           preferred_element_type=jnp.float32)
    est_ref[...] = jax.lax.dot_general(
        abig, wh, (((0,), (1,)), ((), ())),
        precision=jax.lax.Precision.HIGHEST,
        preferred_element_type=jnp.float32)


def _attn_kernel(whl_ref, whr_ref, mask_ref, est_ref, edc_ref, pre_ref,
                 out_ref, *scratch, act):
    accs = scratch[0:HEADS]
    ms = scratch[HEADS:2 * HEADS]
    ds = scratch[2 * HEADS:3 * HEADS]
    j = pl.program_id(1)
    nj = pl.num_programs(1)
    edc = edc_ref[...]

    @pl.when(j == 0)
    def _init():
        whr = whr_ref[...]
        for h in range(HEADS):
            sl = slice(h * HEAD, (h + 1) * HEAD)
            accs[h][...] = whr[:, sl]
            ds[h][...] = jnp.ones_like(ds[h])
            e_self = _leaky(edc[:, HEADS + h:HEADS + h + 1]
                            + edc[:, h:h + 1])
            ms[h][:, 0:1] = e_self

    whl = whl_ref[...]
    TL = whl.shape[0]
    bias = (mask_ref[...].astype(jnp.float32) - 1.0) * (-NEG)
    ones = jnp.ones((TL, 8), jnp.float32)

    for h in range(HEADS):
        sl = slice(h * HEAD, (h + 1) * HEAD)
        ed_r = edc[:, h:h + 1]
        es_l = est_ref[HEADS + h:HEADS + h + 1, :]
        e = _leaky(ed_r + es_l) + bias
        m_old = ms[h][:, 0:1]
        m_new = jnp.maximum(m_old, jnp.max(e, axis=1, keepdims=True))
        scale = jnp.exp2(m_old - m_new)
        ex = jnp.exp2(e - m_new)
        ms[h][:, 0:1] = m_new
        den_inc = jnp.dot(ex, ones, preferred_element_type=jnp.float32)
        ds[h][...] = ds[h][...] * scale + den_inc
        accs[h][...] = accs[h][...] * scale + jnp.dot(
            ex, whl[:, sl], preferred_element_type=jnp.float32)

    @pl.when(j == nj - 1)
    def _finish():
        for h in range(HEADS):
            sl = slice(h * HEAD, (h + 1) * HEAD)
            o = accs[h][...] / (ds[h][:, 0:1] + 1e-9)
            out_ref[:, sl] = _elu(o) if act else o


def _gather_kernel(h_ref, w_ref, asrc_ref, adst_ref, out_ref, *, P):
    hall = h_ref[...]
    Dh = hall.shape[1]
    GB = hall.shape[0] // P
    hbs = [hall[k * P:(k + 1) * P, :] for k in range(GB)]
    for i in range(LAYERS):
        for k in range(GB):
            hb = hbs[k]
            wh = jnp.dot(hb, w_ref[i], preferred_element_type=jnp.float32)
            parts = []
            for h in range(HEADS):
                sl = slice(h * HEAD, (h + 1) * HEAD)
                wh_h = wh[:, sl]
                es = jnp.sum(wh_h * asrc_ref[i, h][None, :], axis=1,
                             keepdims=True)
                ed0 = jnp.sum(wh_h[0:1, :] * adst_ref[i, h][None, :],
                              axis=1, keepdims=True)
                e = _leaky(es + ed0)
                m = jnp.max(e, axis=0, keepdims=True)
                ex = jnp.exp(e - m)
                den = jnp.sum(ex, axis=0, keepdims=True) + ex[0:1, :]
                num = jnp.sum(ex * wh_h, axis=0, keepdims=True) \
                    + ex[0:1, :] * wh_h[0:1, :]
                parts.append(num / (den + 1e-9))
            row0 = jnp.concatenate(parts, axis=1)
            if i < LAYERS - 1:
                body = _elu(wh)
                row0 = _elu(row0)
                rid = jax.lax.broadcasted_iota(jnp.int32, (P, Dh), 0)
                hbs[k] = jnp.where(rid == 0,
                                   jnp.broadcast_to(row0, (P, Dh)), body)
            else:
                out_ref[k, :, :] = row0


def kernel(l_feat, r_feat, W0, asrc0, adst0, W1, asrc1, adst1,
           W2, asrc2, adst2, W3, asrc3, adst3):
    Ws = [W0, W1, W2, W3]
    As = [asrc0, asrc1, asrc2, asrc3]
    Ad = [adst0, adst1, adst2, adst3]
    B, P, Dh = l_feat.shape
    node_l = l_feat.reshape(-1, Dh)
    node_r = r_feat.reshape(-1, Dh)
    L = node_l.shape[0]
    R = node_r.shape[0]
    N = L + R

    TR = min(256, R)
    TL = min(512, L)
    TM = min(512, N)

    h = jnp.concatenate([node_l, node_r], axis=0)

    hn = pl.pallas_call(
        _norm_kernel,
        grid=(N // TM,),
        in_specs=[pl.BlockSpec((TM, Dh), lambda t: (t, 0))],
        out_specs=pl.BlockSpec((TM, Dh), lambda t: (t, 0)),
        out_shape=jax.ShapeDtypeStruct((N, Dh), jnp.float32),
        compiler_params=pltpu.CompilerParams(
            dimension_semantics=("parallel",)),
    )(h)

    rofs = L // TR
    mask = pl.pallas_call(
        _mask_kernel,
        grid=(R // TR, L // TR),
        in_specs=[
            pl.BlockSpec((TR, Dh), lambda i, j, o=rofs: (i + o, 0)),
            pl.BlockSpec((TR, Dh), lambda i, j: (j, 0)),
        ],
        out_specs=pl.BlockSpec((TR, TR), lambda i, j: (i, j)),
        out_shape=jax.ShapeDtypeStruct((R, L), jnp.int8),
        compiler_params=pltpu.CompilerParams(
            dimension_semantics=("parallel", "parallel")),
    )(hn, hn)

    log2e = jnp.float32(1.4426950408889634)
    abigs = []
    for i in range(LAYERS):
        a = jnp.zeros((Dh, 2 * HEADS), jnp.float32)
        for hh in range(HEADS):
            a = a.at[hh * HEAD:(hh + 1) * HEAD, hh].set(Ad[i][hh] * log2e)
            a = a.at[hh * HEAD:(hh + 1) * HEAD,
                     HEADS + hh].set(As[i][hh] * log2e)
        abigs.append(a)

    for i in range(LAYERS):
        act = i < LAYERS - 1
        wh, pre, est, edc = pl.pallas_call(
            functools.partial(_proj_kernel, act=act),
            grid=(N // TM,),
            in_specs=[
                pl.BlockSpec((TM, Dh), lambda t: (t, 0)),
                pl.BlockSpec((Dh, Dh), lambda t: (0, 0)),
                pl.BlockSpec((Dh, 2 * HEADS), lambda t: (0, 0)),
            ],
            out_specs=[
                pl.BlockSpec((TM, Dh), lambda t: (t, 0)),
                pl.BlockSpec((TM, Dh), lambda t: (t, 0)),
                pl.BlockSpec((2 * HEADS, TM), lambda t: (0, t)),
                pl.BlockSpec((TM, 2 * HEADS), lambda t: (t, 0)),
            ],
            out_shape=[
                jax.ShapeDtypeStruct((N, Dh), jnp.float32),
                jax.ShapeDtypeStruct((N, Dh), jnp.float32),
                jax.ShapeDtypeStruct((2 * HEADS, N), jnp.float32),
                jax.ShapeDtypeStruct((N, 2 * HEADS), jnp.float32),
            ],
            compiler_params=pltpu.CompilerParams(
                dimension_semantics=("parallel",)),
        )(h, Ws[i], abigs[i])

        lofs = L // TR
        h = pl.pallas_call(
            functools.partial(_attn_kernel, act=act),
            grid=(R // TR, L // TL),
            in_specs=[
                pl.BlockSpec((TL, Dh), lambda i, j: (j, 0)),
                pl.BlockSpec((TR, Dh), lambda i, j, o=lofs: (i + o, 0)),
                pl.BlockSpec((TR, TL), lambda i, j: (i, j)),
                pl.BlockSpec((2 * HEADS, TL), lambda i, j: (0, j)),
                pl.BlockSpec((TR, 2 * HEADS), lambda i, j, o=lofs: (i + o, 0)),
                pl.BlockSpec((8, 128), lambda i, j: (0, 0)),
            ],
            out_specs=pl.BlockSpec((TR, Dh), lambda i, j, o=lofs: (i + o, 0)),
            out_shape=jax.ShapeDtypeStruct((N, Dh), jnp.float32),
            input_output_aliases={5: 0},
            scratch_shapes=(
                [pltpu.VMEM((TR, HEAD), jnp.float32) for _ in range(HEADS)]
                + [pltpu.VMEM((TR, 8), jnp.float32) for _ in range(HEADS)]
                + [pltpu.VMEM((TR, 8), jnp.float32) for _ in range(HEADS)]
            ),
            compiler_params=pltpu.CompilerParams(
                dimension_semantics=("parallel", "arbitrary")),
        )(wh, wh, mask, est, edc, pre)

    NB = N // P
    GB = min(4, NB)
    wstack = jnp.stack(Ws)
    astack = jnp.stack(As)
    adstack = jnp.stack(Ad)
    g = pl.pallas_call(
        functools.partial(_gather_kernel, P=P),
        grid=(NB // GB,),
        in_specs=[
            pl.BlockSpec((GB * P, Dh), lambda b: (b, 0)),
            pl.BlockSpec((LAYERS, Dh, Dh), lambda b: (0, 0, 0)),
            pl.BlockSpec((LAYERS, HEADS, HEAD), lambda b: (0, 0, 0)),
            pl.BlockSpec((LAYERS, HEADS, HEAD), lambda b: (0, 0, 0)),
        ],
        out_specs=pl.BlockSpec((GB, 1, Dh), lambda b: (b, 0, 0)),
        out_shape=jax.ShapeDtypeStruct((NB, 1, Dh), jnp.float32),
        compiler_params=pltpu.CompilerParams(
            dimension_semantics=("parallel",)),
    )(h, wstack, astack, adstack)

    g = g.reshape(NB, Dh)
    return g[:B], g[B:]


# speedup vs baseline: 8.6457x; 1.0036x over previous
import functools

import jax
import jax.numpy as jnp
from jax.experimental import pallas as pl
from jax.experimental.pallas import tpu as pltpu

HEADS = 4
HEAD = 64
THRESH = 0.15
LAYERS = 4
NEG = -1e30


def _leaky(x):
    return jnp.maximum(x, 0.2 * x)


def _elu(x):
    return jnp.where(x > 0, x, jnp.exp(jnp.minimum(x, 0.0)) - 1.0)


def _norm_kernel(x_ref, o_ref):
    x = x_ref[...]
    o_ref[...] = x / (jnp.sqrt(jnp.sum(x * x, axis=1, keepdims=True)) + 1e-12)


def _mask_kernel(ln_ref, rn_ref, out_ref):
    sim = jax.lax.dot_general(
        ln_ref[...], rn_ref[...], (((1,), (1,)), ((), ())),
        precision=jax.lax.Precision.HIGHEST,
        preferred_element_type=jnp.float32)
    out_ref[...] = (sim > THRESH).astype(jnp.int8)


def _proj_kernel(h_ref, w_ref, abig_ref, wh_ref, pre_ref, est_ref, edc_ref,
                 *, act):
    wh = jnp.dot(h_ref[...], w_ref[...], preferred_element_type=jnp.float32)
    wh_ref[...] = wh
    pre_ref[...] = _elu(wh) if act else wh
    abig = abig_ref[...]
    edc_ref[...] = jnp.dot(wh, abig, precision=jax.lax.Precision.HIGHEST,
                           preferred_element_type=jnp.float32)
    est_ref[...] = jax.lax.dot_general(
        abig, wh, (((0,), (1,)), ((), ())),
        precision=jax.lax.Precision.HIGHEST,
        preferred_element_type=jnp.float32)


def _attn_kernel(whl_ref, whr_ref, mask_ref, est_ref, edc_ref, pre_ref,
                 out_ref, *scratch, act):
    accs = scratch[0:HEADS]
    ms = scratch[HEADS:2 * HEADS]
    ds = scratch[2 * HEADS:3 * HEADS]
    j = pl.program_id(1)
    nj = pl.num_programs(1)
    edl = edc_ref[...]
    estr = est_ref[...]

    @pl.when(j == 0)
    def _init():
        whr = whr_ref[...]
        for h in range(HEADS):
            sl = slice(h * HEAD, (h + 1) * HEAD)
            accs[h][...] = whr[:, sl]
            ds[h][...] = jnp.ones_like(ds[h])
            e_self = _leaky(estr[HEADS + h:HEADS + h + 1, :]
                            + estr[h:h + 1, :])
            ms[h][0:1, :] = e_self

    whl = whl_ref[...]
    TL = whl.shape[0]
    bias = (mask_ref[...].astype(jnp.float32) - 1.0) * (-NEG)
    ones = jnp.ones((TL, 8), jnp.float32)
    dn = (((0,), (0,)), ((), ()))

    for h in range(HEADS):
        sl = slice(h * HEAD, (h + 1) * HEAD)
        es_l = edl[:, HEADS + h:HEADS + h + 1]
        ed_r = estr[h:h + 1, :]
        e = _leaky(es_l + ed_r) + bias
        m_old = ms[h][0:1, :]
        m_new = jnp.maximum(m_old, jnp.max(e, axis=0, keepdims=True))
        scale = jnp.exp2(m_old - m_new)
        ex = jnp.exp2(e - m_new)
        ms[h][0:1, :] = m_new
        scale_c = jnp.transpose(scale)
        den_inc = jax.lax.dot_general(
            ex, ones, dn, preferred_element_type=jnp.float32)
        ds[h][...] = ds[h][...] * scale_c + den_inc
        accs[h][...] = accs[h][...] * scale_c + jax.lax.dot_general(
            ex, whl[:, sl], dn, preferred_element_type=jnp.float32)

    @pl.when(j == nj - 1)
    def _finish():
        for h in range(HEADS):
            sl = slice(h * HEAD, (h + 1) * HEAD)
            o = accs[h][...] / (ds[h][:, 0:1] + 1e-9)
            out_ref[:, sl] = _elu(o) if act else o


def _gather_kernel(h_ref, w_ref, asrc_ref, adst_ref, out_ref, *, P):
    hall = h_ref[...]
    Dh = hall.shape[1]
    GB = hall.shape[0] // P
    hbs = [hall[k * P:(k + 1) * P, :] for k in range(GB)]
    for i in range(LAYERS):
        for k in range(GB):
            hb = hbs[k]
            wh = jnp.dot(hb, w_ref[i], preferred_element_type=jnp.float32)
            parts = []
            for h in range(HEADS):
                sl = slice(h * HEAD, (h + 1) * HEAD)
                wh_h = wh[:, sl]
                es = jnp.sum(wh_h * asrc_ref[i, h][None, :], axis=1,
                             keepdims=True)
                ed0 = jnp.sum(wh_h[0:1, :] * adst_ref[i, h][None, :],
                              axis=1, keepdims=True)
                e = _leaky(es + ed0)
                m = jnp.max(e, axis=0, keepdims=True)
                ex = jnp.exp(e - m)
                den = jnp.sum(ex, axis=0, keepdims=True) + ex[0:1, :]
                num = jnp.sum(ex * wh_h, axis=0, keepdims=True) \
                    + ex[0:1, :] * wh_h[0:1, :]
                parts.append(num / (den + 1e-9))
            row0 = jnp.concatenate(parts, axis=1)
            if i < LAYERS - 1:
                body = _elu(wh)
                row0 = _elu(row0)
                rid = jax.lax.broadcasted_iota(jnp.int32, (P, Dh), 0)
                hbs[k] = jnp.where(rid == 0,
                                   jnp.broadcast_to(row0, (P, Dh)), body)
            else:
                out_ref[k, :, :] = row0


def kernel(l_feat, r_feat, W0, asrc0, adst0, W1, asrc1, adst1,
           W2, asrc2, adst2, W3, asrc3, adst3):
    Ws = [W0, W1, W2, W3]
    As = [asrc0, asrc1, asrc2, asrc3]
    Ad = [adst0, adst1, adst2, adst3]
    B, P, Dh = l_feat.shape
    node_l = l_feat.reshape(-1, Dh)
    node_r = r_feat.reshape(-1, Dh)
    L = node_l.shape[0]
    R = node_r.shape[0]
    N = L + R

    TR = min(256, R)
    TL = min(512, L)
    TM = min(512, N)

    h = jnp.concatenate([node_l, node_r], axis=0)

    hn = pl.pallas_call(
        _norm_kernel,
        grid=(N // TM,),
        in_specs=[pl.BlockSpec((TM, Dh), lambda t: (t, 0))],
        out_specs=pl.BlockSpec((TM, Dh), lambda t: (t, 0)),
        out_shape=jax.ShapeDtypeStruct((N, Dh), jnp.float32),
        compiler_params=pltpu.CompilerParams(
            dimension_semantics=("parallel",)),
    )(h)

    rofs = L // TR
    mask = pl.pallas_call(
        _mask_kernel,
        grid=(L // TR, R // TR),
        in_specs=[
            pl.BlockSpec((TR, Dh), lambda i, j: (i, 0)),
            pl.BlockSpec((TR, Dh), lambda i, j, o=rofs: (j + o, 0)),
        ],
        out_specs=pl.BlockSpec((TR, TR), lambda i, j: (i, j)),
        out_shape=jax.ShapeDtypeStruct((L, R), jnp.int8),
        compiler_params=pltpu.CompilerParams(
            dimension_semantics=("parallel", "parallel")),
    )(hn, hn)

    log2e = jnp.float32(1.4426950408889634)
    abigs = []
    for i in range(LAYERS):
        a = jnp.zeros((Dh, 2 * HEADS), jnp.float32)
        for hh in range(HEADS):
            a = a.at[hh * HEAD:(hh + 1) * HEAD, hh].set(Ad[i][hh] * log2e)
            a = a.at[hh * HEAD:(hh + 1) * HEAD,
                     HEADS + hh].set(As[i][hh] * log2e)
        abigs.append(a)

    for i in range(LAYERS):
        act = i < LAYERS - 1
        wh, pre, est, edc = pl.pallas_call(
            functools.partial(_proj_kernel, act=act),
            grid=(N // TM,),
            in_specs=[
                pl.BlockSpec((TM, Dh), lambda t: (t, 0)),
                pl.BlockSpec((Dh, Dh), lambda t: (0, 0)),
                pl.BlockSpec((Dh, 2 * HEADS), lambda t: (0, 0)),
            ],
            out_specs=[
                pl.BlockSpec((TM, Dh), lambda t: (t, 0)),
                pl.BlockSpec((TM, Dh), lambda t: (t, 0)),
                pl.BlockSpec((2 * HEADS, TM), lambda t: (0, t)),
                pl.BlockSpec((TM, 2 * HEADS), lambda t: (t, 0)),
            ],
            out_shape=[
                jax.ShapeDtypeStruct((N, Dh), jnp.float32),
                jax.ShapeDtypeStruct((N, Dh), jnp.float32),
                jax.ShapeDtypeStruct((2 * HEADS, N), jnp.float32),
                jax.ShapeDtypeStruct((N, 2 * HEADS), jnp.float32),
            ],
            compiler_params=pltpu.CompilerParams(
                dimension_semantics=("parallel",)),
        )(h, Ws[i], abigs[i])

        lofs = L // TR
        h = pl.pallas_call(
            functools.partial(_attn_kernel, act=act),
            grid=(R // TR, L // TL),
            in_specs=[
                pl.BlockSpec((TL, Dh), lambda i, j: (j, 0)),
                pl.BlockSpec((TR, Dh), lambda i, j, o=lofs: (i + o, 0)),
                pl.BlockSpec((TL, TR), lambda i, j: (j, i)),
                pl.BlockSpec((2 * HEADS, TR), lambda i, j, o=lofs: (0, i + o)),
                pl.BlockSpec((TL, 2 * HEADS), lambda i, j: (j, 0)),
                pl.BlockSpec((8, 128), lambda i, j: (0, 0)),
            ],
            out_specs=pl.BlockSpec((TR, Dh), lambda i, j, o=lofs: (i + o, 0)),
            out_shape=jax.ShapeDtypeStruct((N, Dh), jnp.float32),
            input_output_aliases={5: 0},
            scratch_shapes=(
                [pltpu.VMEM((TR, HEAD), jnp.float32) for _ in range(HEADS)]
                + [pltpu.VMEM((8, TR), jnp.float32) for _ in range(HEADS)]
                + [pltpu.VMEM((TR, 8), jnp.float32) for _ in range(HEADS)]
            ),
            compiler_params=pltpu.CompilerParams(
                dimension_semantics=("parallel", "arbitrary")),
        )(wh, wh, mask, est, edc, pre)

    NB = N // P
    GB = min(4, NB)
    wstack = jnp.stack(Ws)
    astack = jnp.stack(As)
    adstack = jnp.stack(Ad)
    g = pl.pallas_call(
        functools.partial(_gather_kernel, P=P),
        grid=(NB // GB,),
        in_specs=[
            pl.BlockSpec((GB * P, Dh), lambda b: (b, 0)),
            pl.BlockSpec((LAYERS, Dh, Dh), lambda b: (0, 0, 0)),
            pl.BlockSpec((LAYERS, HEADS, HEAD), lambda b: (0, 0, 0)),
            pl.BlockSpec((LAYERS, HEADS, HEAD), lambda b: (0, 0, 0)),
        ],
        out_specs=pl.BlockSpec((GB, 1, Dh), lambda b: (b, 0, 0)),
        out_shape=jax.ShapeDtypeStruct((NB, 1, Dh), jnp.float32),
        compiler_params=pltpu.CompilerParams(
            dimension_semantics=("parallel",)),
    )(h, wstack, astack, adstack)

    g = g.reshape(NB, Dh)
    return g[:B], g[B:]


# speedup vs baseline: 10.6200x; 1.2284x over previous
import functools

import jax
import jax.numpy as jnp
from jax.experimental import pallas as pl
from jax.experimental.pallas import tpu as pltpu

HEADS = 4
HEAD = 64
THRESH = 0.15
LAYERS = 4
NEG = -1e30


def _leaky(x):
    return jnp.maximum(x, 0.2 * x)


def _elu(x):
    return jnp.where(x > 0, x, jnp.exp(jnp.minimum(x, 0.0)) - 1.0)


def _norm_kernel(x_ref, o_ref):
    x = x_ref[...]
    o_ref[...] = x / (jnp.sqrt(jnp.sum(x * x, axis=1, keepdims=True)) + 1e-12)


def _mask_kernel(ln_ref, rn_ref, out_ref):
    sim = jax.lax.dot_general(
        ln_ref[...], rn_ref[...], (((1,), (1,)), ((), ())),
        precision=jax.lax.Precision.HIGHEST,
        preferred_element_type=jnp.float32)
    out_ref[...] = (sim > THRESH).astype(jnp.int8)


def _proj_kernel(h_ref, w_ref, abig_ref, wh_ref, pre_ref, est_ref, edc_ref,
                 *, act):
    wh = jnp.dot(h_ref[...], w_ref[...], preferred_element_type=jnp.float32)
    wh_ref[...] = wh
    pre_ref[...] = _elu(wh) if act else wh
    abig = abig_ref[...]
    edc_ref[...] = jnp.dot(wh, abig, precision=jax.lax.Precision.HIGHEST,
                           preferred_element_type=jnp.float32)
    est_ref[...] = jax.lax.dot_general(
        abig, wh, (((0,), (1,)), ((), ())),
        precision=jax.lax.Precision.HIGHEST,
        preferred_element_type=jnp.float32)


def _attn_kernel(whl_ref, whr_ref, mask_ref, est_ref, edc_ref, pre_ref,
                 out_ref, *scratch, act):
    accs = scratch[0:HEADS]
    ms = scratch[HEADS:2 * HEADS]
    ds = scratch[2 * HEADS:3 * HEADS]
    j = pl.program_id(1)
    nj = pl.num_programs(1)
    edl = edc_ref[...]
    estr = est_ref[...]

    @pl.when(j == 0)
    def _init():
        whr = whr_ref[...]
        for h in range(HEADS):
            sl = slice(h * HEAD, (h + 1) * HEAD)
            accs[h][...] = jnp.transpose(whr[:, sl])
            ds[h][...] = jnp.ones_like(ds[h])
            e_self = _leaky(estr[HEADS + h:HEADS + h + 1, :]
                            + estr[h:h + 1, :])
            ms[h][0:1, :] = e_self

    whl = whl_ref[...]
    TL = whl.shape[0]
    bias = (mask_ref[...].astype(jnp.float32) - 1.0) * (-NEG)
    ones = jnp.ones((TL, 8), jnp.float32)
    dn = (((0,), (0,)), ((), ()))

    TLC = 128
    nc = TL // TLC
    for h in range(HEADS):
        sl = slice(h * HEAD, (h + 1) * HEAD)
        ed_r = estr[h:h + 1, :]
        ts = []
        m_blk = None
        for c in range(nc):
            cs = slice(c * TLC, (c + 1) * TLC)
            es_c = edl[cs, HEADS + h:HEADS + h + 1]
            t = _leaky(es_c + ed_r) + bias[cs, :]
            ts.append(t)
            mc = jnp.max(t, axis=0, keepdims=True)
            m_blk = mc if m_blk is None else jnp.maximum(m_blk, mc)
        m_old = ms[h][0:1, :]
        m_new = jnp.maximum(m_old, m_blk)
        scale = jnp.exp2(m_old - m_new)
        ms[h][0:1, :] = m_new
        den_inc = None
        acc_inc = None
        for c in range(nc):
            cs = slice(c * TLC, (c + 1) * TLC)
            ex = jnp.exp2(ts[c] - m_new)
            di = jax.lax.dot_general(
                ones[cs, :], ex, dn, preferred_element_type=jnp.float32)
            ai = jax.lax.dot_general(
                whl[cs, sl], ex, dn, preferred_element_type=jnp.float32)
            den_inc = di if den_inc is None else den_inc + di
            acc_inc = ai if acc_inc is None else acc_inc + ai
        ds[h][...] = ds[h][...] * scale + den_inc
        accs[h][...] = accs[h][...] * scale + acc_inc

    @pl.when(j == nj - 1)
    def _finish():
        for h in range(HEADS):
            sl = slice(h * HEAD, (h + 1) * HEAD)
            o = accs[h][...] / (ds[h][0:1, :] + 1e-9)
            out_ref[:, sl] = jnp.transpose(_elu(o) if act else o)


def _gather_kernel(h_ref, w_ref, asrc_ref, adst_ref, out_ref, *, P):
    hall = h_ref[...]
    Dh = hall.shape[1]
    GB = hall.shape[0] // P
    hbs = [hall[k * P:(k + 1) * P, :] for k in range(GB)]
    for i in range(LAYERS):
        for k in range(GB):
            hb = hbs[k]
            wh = jnp.dot(hb, w_ref[i], preferred_element_type=jnp.float32)
            parts = []
            for h in range(HEADS):
                sl = slice(h * HEAD, (h + 1) * HEAD)
                wh_h = wh[:, sl]
                es = jnp.sum(wh_h * asrc_ref[i, h][None, :], axis=1,
                             keepdims=True)
                ed0 = jnp.sum(wh_h[0:1, :] * adst_ref[i, h][None, :],
                              axis=1, keepdims=True)
                e = _leaky(es + ed0)
                m = jnp.max(e, axis=0, keepdims=True)
                ex = jnp.exp(e - m)
                den = jnp.sum(ex, axis=0, keepdims=True) + ex[0:1, :]
                num = jnp.sum(ex * wh_h, axis=0, keepdims=True) \
                    + ex[0:1, :] * wh_h[0:1, :]
                parts.append(num / (den + 1e-9))
            row0 = jnp.concatenate(parts, axis=1)
            if i < LAYERS - 1:
                body = _elu(wh)
                row0 = _elu(row0)
                rid = jax.lax.broadcasted_iota(jnp.int32, (P, Dh), 0)
                hbs[k] = jnp.where(rid == 0,
                                   jnp.broadcast_to(row0, (P, Dh)), body)
            else:
                out_ref[k, :, :] = row0


def kernel(l_feat, r_feat, W0, asrc0, adst0, W1, asrc1, adst1,
           W2, asrc2, adst2, W3, asrc3, adst3):
    Ws = [W0, W1, W2, W3]
    As = [asrc0, asrc1, asrc2, asrc3]
    Ad = [adst0, adst1, adst2, adst3]
    B, P, Dh = l_feat.shape
    node_l = l_feat.reshape(-1, Dh)
    node_r = r_feat.reshape(-1, Dh)
    L = node_l.shape[0]
    R = node_r.shape[0]
    N = L + R

    TR = min(256, R)
    TL = min(512, L)
    TM = min(512, N)

    h = jnp.concatenate([node_l, node_r], axis=0)

    hn = pl.pallas_call(
        _norm_kernel,
        grid=(N // TM,),
        in_specs=[pl.BlockSpec((TM, Dh), lambda t: (t, 0))],
        out_specs=pl.BlockSpec((TM, Dh), lambda t: (t, 0)),
        out_shape=jax.ShapeDtypeStruct((N, Dh), jnp.float32),
        compiler_params=pltpu.CompilerParams(
            dimension_semantics=("parallel",)),
    )(h)

    rofs = L // TR
    mask = pl.pallas_call(
        _mask_kernel,
        grid=(L // TR, R // TR),
        in_specs=[
            pl.BlockSpec((TR, Dh), lambda i, j: (i, 0)),
            pl.BlockSpec((TR, Dh), lambda i, j, o=rofs: (j + o, 0)),
        ],
        out_specs=pl.BlockSpec((TR, TR), lambda i, j: (i, j)),
        out_shape=jax.ShapeDtypeStruct((L, R), jnp.int8),
        compiler_params=pltpu.CompilerParams(
            dimension_semantics=("parallel", "parallel")),
    )(hn, hn)

    log2e = jnp.float32(1.4426950408889634)
    abigs = []
    for i in range(LAYERS):
        a = jnp.zeros((Dh, 2 * HEADS), jnp.float32)
        for hh in range(HEADS):
            a = a.at[hh * HEAD:(hh + 1) * HEAD, hh].set(Ad[i][hh] * log2e)
            a = a.at[hh * HEAD:(hh + 1) * HEAD,
                     HEADS + hh].set(As[i][hh] * log2e)
        abigs.append(a)

    for i in range(LAYERS):
        act = i < LAYERS - 1
        wh, pre, est, edc = pl.pallas_call(
            functools.partial(_proj_kernel, act=act),
            grid=(N // TM,),
            in_specs=[
                pl.BlockSpec((TM, Dh), lambda t: (t, 0)),
                pl.BlockSpec((Dh, Dh), lambda t: (0, 0)),
                pl.BlockSpec((Dh, 2 * HEADS), lambda t: (0, 0)),
            ],
            out_specs=[
                pl.BlockSpec((TM, Dh), lambda t: (t, 0)),
                pl.BlockSpec((TM, Dh), lambda t: (t, 0)),
                pl.BlockSpec((2 * HEADS, TM), lambda t: (0, t)),
                pl.BlockSpec((TM, 2 * HEADS), lambda t: (t, 0)),
            ],
            out_shape=[
                jax.ShapeDtypeStruct((N, Dh), jnp.float32),
                jax.ShapeDtypeStruct((N, Dh), jnp.float32),
                jax.ShapeDtypeStruct((2 * HEADS, N), jnp.float32),
                jax.ShapeDtypeStruct((N, 2 * HEADS), jnp.float32),
            ],
            compiler_params=pltpu.CompilerParams(
                dimension_semantics=("parallel",)),
        )(h, Ws[i], abigs[i])

        lofs = L // TR
        h = pl.pallas_call(
            functools.partial(_attn_kernel, act=act),
            grid=(R // TR, L // TL),
            in_specs=[
                pl.BlockSpec((TL, Dh), lambda i, j: (j, 0)),
                pl.BlockSpec((TR, Dh), lambda i, j, o=lofs: (i + o, 0)),
                pl.BlockSpec((TL, TR), lambda i, j: (j, i)),
                pl.BlockSpec((2 * HEADS, TR), lambda i, j, o=lofs: (0, i + o)),
                pl.BlockSpec((TL, 2 * HEADS), lambda i, j: (j, 0)),
                pl.BlockSpec((8, 128), lambda i, j: (0, 0)),
            ],
            out_specs=pl.BlockSpec((TR, Dh), lambda i, j, o=lofs: (i + o, 0)),
            out_shape=jax.ShapeDtypeStruct((N, Dh), jnp.float32),
            input_output_aliases={5: 0},
            scratch_shapes=(
                [pltpu.VMEM((HEAD, TR), jnp.float32) for _ in range(HEADS)]
                + [pltpu.VMEM((8, TR), jnp.float32) for _ in range(HEADS)]
                + [pltpu.VMEM((8, TR), jnp.float32) for _ in range(HEADS)]
            ),
            compiler_params=pltpu.CompilerParams(
                dimension_semantics=("parallel", "arbitrary")),
        )(wh, wh, mask, est, edc, pre)

    NB = N // P
    GB = min(4, NB)
    wstack = jnp.stack(Ws)
    astack = jnp.stack(As)
    adstack = jnp.stack(Ad)
    g = pl.pallas_call(
        functools.partial(_gather_kernel, P=P),
        grid=(NB // GB,),
        in_specs=[
            pl.BlockSpec((GB * P, Dh), lambda b: (b, 0)),
            pl.BlockSpec((LAYERS, Dh, Dh), lambda b: (0, 0, 0)),
            pl.BlockSpec((LAYERS, HEADS, HEAD), lambda b: (0, 0, 0)),
            pl.BlockSpec((LAYERS, HEADS, HEAD), lambda b: (0, 0, 0)),
        ],
        out_specs=pl.BlockSpec((GB, 1, Dh), lambda b: (b, 0, 0)),
        out_shape=jax.ShapeDtypeStruct((NB, 1, Dh), jnp.float32),
        compiler_params=pltpu.CompilerParams(
            dimension_semantics=("parallel",)),
    )(h, wstack, astack, adstack)

    g = g.reshape(NB, Dh)
    return g[:B], g[B:]


# speedup vs baseline: 11.4097x; 1.0744x over previous
import functools

import jax
import jax.numpy as jnp
from jax.experimental import pallas as pl
from jax.experimental.pallas import tpu as pltpu

HEADS = 4
HEAD = 64
THRESH = 0.15
LAYERS = 4
NEG = -1e30


def _leaky(x):
    return jnp.maximum(x, 0.2 * x)


def _elu(x):
    return jnp.where(x > 0, x, jnp.exp(jnp.minimum(x, 0.0)) - 1.0)


def _norm_kernel(x_ref, hi_ref, lo_ref):
    x = x_ref[...]
    n = x / (jnp.sqrt(jnp.sum(x * x, axis=1, keepdims=True)) + 1e-12)
    hi = n.astype(jnp.bfloat16)
    hi_ref[...] = hi
    lo_ref[...] = (n - hi.astype(jnp.float32)).astype(jnp.bfloat16)


def _mask_kernel(lhi_ref, llo_ref, rhi_ref, rlo_ref, out_ref):
    dnum = (((1,), (1,)), ((), ()))
    lhi = lhi_ref[...]
    rhi = rhi_ref[...]
    sim = (jax.lax.dot_general(lhi, rhi, dnum,
                               preferred_element_type=jnp.float32)
           + jax.lax.dot_general(lhi, rlo_ref[...], dnum,
                                 preferred_element_type=jnp.float32)
           + jax.lax.dot_general(llo_ref[...], rhi, dnum,
                                 preferred_element_type=jnp.float32))
    out_ref[...] = (sim > THRESH).astype(jnp.int8)


def _proj_kernel(h_ref, w_ref, abig_ref, wh_ref, pre_ref, est_ref, edc_ref,
                 *, act):
    wh = jnp.dot(h_ref[...], w_ref[...], preferred_element_type=jnp.float32)
    wh_ref[...] = wh
    pre_ref[...] = _elu(wh) if act else wh
    abig = abig_ref[...]
    edc_ref[...] = jnp.dot(wh, abig, precision=jax.lax.Precision.HIGHEST,
                           preferred_element_type=jnp.float32)
    est_ref[...] = jax.lax.dot_general(
        abig, wh, (((0,), (1,)), ((), ())),
        precision=jax.lax.Precision.HIGHEST,
        preferred_element_type=jnp.float32)


def _attn_kernel(whl_ref, whr_ref, mask_ref, est_ref, edc_ref, pre_ref,
                 out_ref, *scratch, act):
    accs = scratch[0:HEADS]
    ms = scratch[HEADS:2 * HEADS]
    ds = scratch[2 * HEADS:3 * HEADS]
    j = pl.program_id(1)
    nj = pl.num_programs(1)
    edl = edc_ref[...]
    estr = est_ref[...]

    @pl.when(j == 0)
    def _init():
        whr = whr_ref[...]
        for h in range(HEADS):
            sl = slice(h * HEAD, (h + 1) * HEAD)
            accs[h][...] = jnp.transpose(whr[:, sl])
            ds[h][...] = jnp.ones_like(ds[h])
            e_self = _leaky(estr[HEADS + h:HEADS + h + 1, :]
                            + estr[h:h + 1, :])
            ms[h][0:1, :] = e_self

    whl = whl_ref[...]
    TL = whl.shape[0]
    bias = (mask_ref[...].astype(jnp.float32) - 1.0) * (-NEG)
    ones = jnp.ones((TL, 8), jnp.float32)
    dn = (((0,), (0,)), ((), ()))

    TLC = 128
    nc = TL // TLC
    for h in range(HEADS):
        sl = slice(h * HEAD, (h + 1) * HEAD)
        ed_r = estr[h:h + 1, :]
        ts = []
        m_blk = None
        for c in range(nc):
            cs = slice(c * TLC, (c + 1) * TLC)
            es_c = edl[cs, HEADS + h:HEADS + h + 1]
            t = _leaky(es_c + ed_r) + bias[cs, :]
            ts.append(t)
            mc = jnp.max(t, axis=0, keepdims=True)
            m_blk = mc if m_blk is None else jnp.maximum(m_blk, mc)
        m_old = ms[h][0:1, :]
        m_new = jnp.maximum(m_old, m_blk)
        scale = jnp.exp2(m_old - m_new)
        ms[h][0:1, :] = m_new
        den_inc = None
        acc_inc = None
        for c in range(nc):
            cs = slice(c * TLC, (c + 1) * TLC)
            ex = jnp.exp2(ts[c] - m_new)
            di = jax.lax.dot_general(
                ones[cs, :], ex, dn, preferred_element_type=jnp.float32)
            ai = jax.lax.dot_general(
                whl[cs, sl], ex, dn, preferred_element_type=jnp.float32)
            den_inc = di if den_inc is None else den_inc + di
            acc_inc = ai if acc_inc is None else acc_inc + ai
        ds[h][...] = ds[h][...] * scale + den_inc
        accs[h][...] = accs[h][...] * scale + acc_inc

    @pl.when(j == nj - 1)
    def _finish():
        for h in range(HEADS):
            sl = slice(h * HEAD, (h + 1) * HEAD)
            o = accs[h][...] / (ds[h][0:1, :] + 1e-9)
            out_ref[:, sl] = jnp.transpose(_elu(o) if act else o)


def _gather_kernel(h_ref, w_ref, asrc_ref, adst_ref, out_ref, *, P):
    hall = h_ref[...]
    Dh = hall.shape[1]
    GB = hall.shape[0] // P
    hbs = [hall[k * P:(k + 1) * P, :] for k in range(GB)]
    for i in range(LAYERS):
        for k in range(GB):
            hb = hbs[k]
            wh = jnp.dot(hb, w_ref[i], preferred_element_type=jnp.float32)
            parts = []
            for h in range(HEADS):
                sl = slice(h * HEAD, (h + 1) * HEAD)
                wh_h = wh[:, sl]
                es = jnp.sum(wh_h * asrc_ref[i, h][None, :], axis=1,
                             keepdims=True)
                ed0 = jnp.sum(wh_h[0:1, :] * adst_ref[i, h][None, :],
                              axis=1, keepdims=True)
                e = _leaky(es + ed0)
                m = jnp.max(e, axis=0, keepdims=True)
                ex = jnp.exp(e - m)
                den = jnp.sum(ex, axis=0, keepdims=True) + ex[0:1, :]
                num = jnp.sum(ex * wh_h, axis=0, keepdims=True) \
                    + ex[0:1, :] * wh_h[0:1, :]
                parts.append(num / (den + 1e-9))
            row0 = jnp.concatenate(parts, axis=1)
            if i < LAYERS - 1:
                body = _elu(wh)
                row0 = _elu(row0)
                rid = jax.lax.broadcasted_iota(jnp.int32, (P, Dh), 0)
                hbs[k] = jnp.where(rid == 0,
                                   jnp.broadcast_to(row0, (P, Dh)), body)
            else:
                out_ref[k, :, :] = row0


def kernel(l_feat, r_feat, W0, asrc0, adst0, W1, asrc1, adst1,
           W2, asrc2, adst2, W3, asrc3, adst3):
    Ws = [W0, W1, W2, W3]
    As = [asrc0, asrc1, asrc2, asrc3]
    Ad = [adst0, adst1, adst2, adst3]
    B, P, Dh = l_feat.shape
    node_l = l_feat.reshape(-1, Dh)
    node_r = r_feat.reshape(-1, Dh)
    L = node_l.shape[0]
    R = node_r.shape[0]
    N = L + R

    TR = min(256, R)
    TL = min(1024, L)
    TM = min(512, N)

    h = jnp.concatenate([node_l, node_r], axis=0)

    nhi, nlo = pl.pallas_call(
        _norm_kernel,
        grid=(N // TM,),
        in_specs=[pl.BlockSpec((TM, Dh), lambda t: (t, 0))],
        out_specs=[
            pl.BlockSpec((TM, Dh), lambda t: (t, 0)),
            pl.BlockSpec((TM, Dh), lambda t: (t, 0)),
        ],
        out_shape=[
            jax.ShapeDtypeStruct((N, Dh), jnp.bfloat16),
            jax.ShapeDtypeStruct((N, Dh), jnp.bfloat16),
        ],
        compiler_params=pltpu.CompilerParams(
            dimension_semantics=("parallel",)),
    )(h)

    rofs = L // TR
    mask = pl.pallas_call(
        _mask_kernel,
        grid=(L // TR, R // TR),
        in_specs=[
            pl.BlockSpec((TR, Dh), lambda i, j: (i, 0)),
            pl.BlockSpec((TR, Dh), lambda i, j: (i, 0)),
            pl.BlockSpec((TR, Dh), lambda i, j, o=rofs: (j + o, 0)),
            pl.BlockSpec((TR, Dh), lambda i, j, o=rofs: (j + o, 0)),
        ],
        out_specs=pl.BlockSpec((TR, TR), lambda i, j: (i, j)),
        out_shape=jax.ShapeDtypeStruct((L, R), jnp.int8),
        compiler_params=pltpu.CompilerParams(
            dimension_semantics=("parallel", "parallel")),
    )(nhi, nlo, nhi, nlo)

    log2e = jnp.float32(1.4426950408889634)
    abigs = []
    for i in range(LAYERS):
        a = jnp.zeros((Dh, 2 * HEADS), jnp.float32)
        for hh in range(HEADS):
            a = a.at[hh * HEAD:(hh + 1) * HEAD, hh].set(Ad[i][hh] * log2e)
            a = a.at[hh * HEAD:(hh + 1) * HEAD,
                     HEADS + hh].set(As[i][hh] * log2e)
        abigs.append(a)

    for i in range(LAYERS):
        act = i < LAYERS - 1
        wh, pre, est, edc = pl.pallas_call(
            functools.partial(_proj_kernel, act=act),
            grid=(N // TM,),
            in_specs=[
                pl.BlockSpec((TM, Dh), lambda t: (t, 0)),
                pl.BlockSpec((Dh, Dh), lambda t: (0, 0)),
                pl.BlockSpec((Dh, 2 * HEADS), lambda t: (0, 0)),
            ],
            out_specs=[
                pl.BlockSpec((TM, Dh), lambda t: (t, 0)),
                pl.BlockSpec((TM, Dh), lambda t: (t, 0)),
                pl.BlockSpec((2 * HEADS, TM), lambda t: (0, t)),
                pl.BlockSpec((TM, 2 * HEADS), lambda t: (t, 0)),
            ],
            out_shape=[
                jax.ShapeDtypeStruct((N, Dh), jnp.float32),
                jax.ShapeDtypeStruct((N, Dh), jnp.float32),
                jax.ShapeDtypeStruct((2 * HEADS, N), jnp.float32),
                jax.ShapeDtypeStruct((N, 2 * HEADS), jnp.float32),
            ],
            compiler_params=pltpu.CompilerParams(
                dimension_semantics=("parallel",)),
        )(h, Ws[i], abigs[i])

        lofs = L // TR
        h = pl.pallas_call(
            functools.partial(_attn_kernel, act=act),
            grid=(R // TR, L // TL),
            in_specs=[
                pl.BlockSpec((TL, Dh), lambda i, j: (j, 0)),
                pl.BlockSpec((TR, Dh), lambda i, j, o=lofs: (i + o, 0)),
                pl.BlockSpec((TL, TR), lambda i, j: (j, i)),
                pl.BlockSpec((2 * HEADS, TR), lambda i, j, o=lofs: (0, i + o)),
                pl.BlockSpec((TL, 2 * HEADS), lambda i, j: (j, 0)),
                pl.BlockSpec((8, 128), lambda i, j: (0, 0)),
            ],
            out_specs=pl.BlockSpec((TR, Dh), lambda i, j, o=lofs: (i + o, 0)),
            out_shape=jax.ShapeDtypeStruct((N, Dh), jnp.float32),
            input_output_aliases={5: 0},
            scratch_shapes=(
                [pltpu.VMEM((HEAD, TR), jnp.float32) for _ in range(HEADS)]
                + [pltpu.VMEM((8, TR), jnp.float32) for _ in range(HEADS)]
                + [pltpu.VMEM((8, TR), jnp.float32) for _ in range(HEADS)]
            ),
            compiler_params=pltpu.CompilerParams(
                dimension_semantics=("parallel", "arbitrary")),
        )(wh, wh, mask, est, edc, pre)

    NB = N // P
    GB = min(8, NB)
    wstack = jnp.stack(Ws)
    astack = jnp.stack(As)
    adstack = jnp.stack(Ad)
    g = pl.pallas_call(
        functools.partial(_gather_kernel, P=P),
        grid=(NB // GB,),
        in_specs=[
            pl.BlockSpec((GB * P, Dh), lambda b: (b, 0)),
            pl.BlockSpec((LAYERS, Dh, Dh), lambda b: (0, 0, 0)),
            pl.BlockSpec((LAYERS, HEADS, HEAD), lambda b: (0, 0, 0)),
            pl.BlockSpec((LAYERS, HEADS, HEAD), lambda b: (0, 0, 0)),
        ],
        out_specs=pl.BlockSpec((GB, 1, Dh), lambda b: (b, 0, 0)),
        out_shape=jax.ShapeDtypeStruct((NB, 1, Dh), jnp.float32),
        compiler_params=pltpu.CompilerParams(
            dimension_semantics=("parallel",)),
    )(h, wstack, astack, adstack)

    g = g.reshape(NB, Dh)
    return g[:B], g[B:]


# speedup vs baseline: 11.7704x; 1.0316x over previous
import functools

import jax
import jax.numpy as jnp
from jax.experimental import pallas as pl
from jax.experimental.pallas import tpu as pltpu

HEADS = 4
HEAD = 64
THRESH = 0.15
LAYERS = 4
NEG = -1e30


def _leaky(x):
    return jnp.maximum(x, 0.2 * x)


def _elu(x):
    return jnp.where(x > 0, x, jnp.exp(jnp.minimum(x, 0.0)) - 1.0)


def _norm_kernel(x_ref, hi_ref, lo_ref):
    x = x_ref[...]
    n = x / (jnp.sqrt(jnp.sum(x * x, axis=1, keepdims=True)) + 1e-12)
    hi = n.astype(jnp.bfloat16)
    hi_ref[...] = hi
    lo_ref[...] = (n - hi.astype(jnp.float32)).astype(jnp.bfloat16)


def _mask_kernel(lhi_ref, llo_ref, rhi_ref, rlo_ref, out_ref):
    dnum = (((1,), (1,)), ((), ()))
    lhi = lhi_ref[...]
    rhi = rhi_ref[...]
    sim = (jax.lax.dot_general(lhi, rhi, dnum,
                               preferred_element_type=jnp.float32)
           + jax.lax.dot_general(lhi, rlo_ref[...], dnum,
                                 preferred_element_type=jnp.float32)
           + jax.lax.dot_general(llo_ref[...], rhi, dnum,
                                 preferred_element_type=jnp.float32))
    out_ref[...] = (sim > THRESH).astype(jnp.int8)


def _proj_kernel(h_ref, w_ref, abig_ref, wh_ref, pre_ref, est_ref, edc_ref,
                 *, act):
    wh = jnp.dot(h_ref[...], w_ref[...], preferred_element_type=jnp.float32)
    wh_ref[...] = wh
    pre_ref[...] = _elu(wh) if act else wh
    abig = abig_ref[...]
    edc_ref[...] = jnp.dot(wh, abig, precision=jax.lax.Precision.HIGHEST,
                           preferred_element_type=jnp.float32)
    est_ref[...] = jax.lax.dot_general(
        abig, wh, (((0,), (1,)), ((), ())),
        precision=jax.lax.Precision.HIGHEST,
        preferred_element_type=jnp.float32)


def _attn_kernel(whl_ref, whr_ref, mask_ref, est_ref, edc_ref, pre_ref,
                 out_ref, *scratch, act):
    accs = scratch[0:HEADS]
    ms = scratch[HEADS:2 * HEADS]
    ds = scratch[2 * HEADS:3 * HEADS]
    j = pl.program_id(1)
    nj = pl.num_programs(1)
    edl = edc_ref[...]
    estr = est_ref[...]

    @pl.when(j == 0)
    def _init():
        whr = whr_ref[...]
        for h in range(HEADS):
            sl = slice(h * HEAD, (h + 1) * HEAD)
            accs[h][...] = jnp.transpose(whr[:, sl])
            ds[h][...] = jnp.ones_like(ds[h])
            e_self = _leaky(estr[HEADS + h:HEADS + h + 1, :]
                            + estr[h:h + 1, :])
            ms[h][0:1, :] = e_self

    whl = whl_ref[...]
    TL = whl.shape[0]
    bias = (mask_ref[...].astype(jnp.float32) - 1.0) * (-NEG)
    ones = jnp.ones((TL, 8), jnp.float32)
    dn = (((0,), (0,)), ((), ()))

    TLC = 128
    nc = TL // TLC
    for h in range(HEADS):
        sl = slice(h * HEAD, (h + 1) * HEAD)
        ed_r = estr[h:h + 1, :]
        ts = []
        m_blk = None
        for c in range(nc):
            cs = slice(c * TLC, (c + 1) * TLC)
            es_c = edl[cs, HEADS + h:HEADS + h + 1]
            t = _leaky(es_c + ed_r) + bias[cs, :]
            ts.append(t)
            mc = jnp.max(t, axis=0, keepdims=True)
            m_blk = mc if m_blk is None else jnp.maximum(m_blk, mc)
        m_old = ms[h][0:1, :]
        m_new = jnp.maximum(m_old, m_blk)
        scale = jnp.exp2(m_old - m_new)
        ms[h][0:1, :] = m_new
        den_inc = None
        acc_inc = None
        for c in range(nc):
            cs = slice(c * TLC, (c + 1) * TLC)
            ex = jnp.exp2(ts[c] - m_new)
            di = jax.lax.dot_general(
                ones[cs, :], ex, dn, preferred_element_type=jnp.float32)
            ai = jax.lax.dot_general(
                whl[cs, sl], ex, dn, preferred_element_type=jnp.float32)
            den_inc = di if den_inc is None else den_inc + di
            acc_inc = ai if acc_inc is None else acc_inc + ai
        ds[h][...] = ds[h][...] * scale + den_inc
        accs[h][...] = accs[h][...] * scale + acc_inc

    @pl.when(j == nj - 1)
    def _finish():
        for h in range(HEADS):
            sl = slice(h * HEAD, (h + 1) * HEAD)
            o = accs[h][...] / (ds[h][0:1, :] + 1e-9)
            out_ref[:, sl] = jnp.transpose(_elu(o) if act else o)


def _gather_kernel(h_ref, w_ref, asrc_ref, adst_ref, out_ref, *, P):
    hall = h_ref[...]
    Dh = hall.shape[1]
    GB = hall.shape[0] // P
    hbs = [hall[k * P:(k + 1) * P, :] for k in range(GB)]
    for i in range(LAYERS):
        for k in range(GB):
            hb = hbs[k]
            wh = jnp.dot(hb, w_ref[i], preferred_element_type=jnp.float32)
            parts = []
            for h in range(HEADS):
                sl = slice(h * HEAD, (h + 1) * HEAD)
                wh_h = wh[:, sl]
                es = jnp.sum(wh_h * asrc_ref[i, h][None, :], axis=1,
                             keepdims=True)
                ed0 = jnp.sum(wh_h[0:1, :] * adst_ref[i, h][None, :],
                              axis=1, keepdims=True)
                e = _leaky(es + ed0)
                m = jnp.max(e, axis=0, keepdims=True)
                ex = jnp.exp(e - m)
                den = jnp.sum(ex, axis=0, keepdims=True) + ex[0:1, :]
                num = jnp.sum(ex * wh_h, axis=0, keepdims=True) \
                    + ex[0:1, :] * wh_h[0:1, :]
                parts.append(num / (den + 1e-9))
            row0 = jnp.concatenate(parts, axis=1)
            if i < LAYERS - 1:
                body = _elu(wh)
                row0 = _elu(row0)
                rid = jax.lax.broadcasted_iota(jnp.int32, (P, Dh), 0)
                hbs[k] = jnp.where(rid == 0,
                                   jnp.broadcast_to(row0, (P, Dh)), body)
            else:
                out_ref[k, :, :] = row0


def kernel(l_feat, r_feat, W0, asrc0, adst0, W1, asrc1, adst1,
           W2, asrc2, adst2, W3, asrc3, adst3):
    Ws = [W0, W1, W2, W3]
    As = [asrc0, asrc1, asrc2, asrc3]
    Ad = [adst0, adst1, adst2, adst3]
    B, P, Dh = l_feat.shape
    node_l = l_feat.reshape(-1, Dh)
    node_r = r_feat.reshape(-1, Dh)
    L = node_l.shape[0]
    R = node_r.shape[0]
    N = L + R

    TR = min(256, R)
    TL = min(1024, L)
    TM = min(1024, N)

    h = jnp.concatenate([node_l, node_r], axis=0)

    nhi, nlo = pl.pallas_call(
        _norm_kernel,
        grid=(N // TM,),
        in_specs=[pl.BlockSpec((TM, Dh), lambda t: (t, 0))],
        out_specs=[
            pl.BlockSpec((TM, Dh), lambda t: (t, 0)),
            pl.BlockSpec((TM, Dh), lambda t: (t, 0)),
        ],
        out_shape=[
            jax.ShapeDtypeStruct((N, Dh), jnp.bfloat16),
            jax.ShapeDtypeStruct((N, Dh), jnp.bfloat16),
        ],
        compiler_params=pltpu.CompilerParams(
            dimension_semantics=("parallel",)),
    )(h)

    rofs = L // TR
    mask = pl.pallas_call(
        _mask_kernel,
        grid=(L // TR, R // TR),
        in_specs=[
            pl.BlockSpec((TR, Dh), lambda i, j: (i, 0)),
            pl.BlockSpec((TR, Dh), lambda i, j: (i, 0)),
            pl.BlockSpec((TR, Dh), lambda i, j, o=rofs: (j + o, 0)),
            pl.BlockSpec((TR, Dh), lambda i, j, o=rofs: (j + o, 0)),
        ],
        out_specs=pl.BlockSpec((TR, TR), lambda i, j: (i, j)),
        out_shape=jax.ShapeDtypeStruct((L, R), jnp.int8),
        compiler_params=pltpu.CompilerParams(
            dimension_semantics=("parallel", "parallel")),
    )(nhi, nlo, nhi, nlo)

    log2e = jnp.float32(1.4426950408889634)
    abigs = []
    for i in range(LAYERS):
        a = jnp.zeros((Dh, 2 * HEADS), jnp.float32)
        for hh in range(HEADS):
            a = a.at[hh * HEAD:(hh + 1) * HEAD, hh].set(Ad[i][hh] * log2e)
            a = a.at[hh * HEAD:(hh + 1) * HEAD,
                     HEADS + hh].set(As[i][hh] * log2e)
        abigs.append(a)

    for i in range(LAYERS):
        act = i < LAYERS - 1
        wh, pre, est, edc = pl.pallas_call(
            functools.partial(_proj_kernel, act=act),
            grid=(N // TM,),
            in_specs=[
                pl.BlockSpec((TM, Dh), lambda t: (t, 0)),
                pl.BlockSpec((Dh, Dh), lambda t: (0, 0)),
                pl.BlockSpec((Dh, 2 * HEADS), lambda t: (0, 0)),
            ],
            out_specs=[
                pl.BlockSpec((TM, Dh), lambda t: (t, 0)),
                pl.BlockSpec((TM, Dh), lambda t: (t, 0)),
                pl.BlockSpec((2 * HEADS, TM), lambda t: (0, t)),
                pl.BlockSpec((TM, 2 * HEADS), lambda t: (t, 0)),
            ],
            out_shape=[
                jax.ShapeDtypeStruct((N, Dh), jnp.float32),
                jax.ShapeDtypeStruct((N, Dh), jnp.float32),
                jax.ShapeDtypeStruct((2 * HEADS, N), jnp.float32),
                jax.ShapeDtypeStruct((N, 2 * HEADS), jnp.float32),
            ],
            compiler_params=pltpu.CompilerParams(
                dimension_semantics=("parallel",)),
        )(h, Ws[i], abigs[i])

        lofs = L // TR
        h = pl.pallas_call(
            functools.partial(_attn_kernel, act=act),
            grid=(R // TR, L // TL),
            in_specs=[
                pl.BlockSpec((TL, Dh), lambda i, j: (j, 0)),
                pl.BlockSpec((TR, Dh), lambda i, j, o=lofs: (i + o, 0)),
                pl.BlockSpec((TL, TR), lambda i, j: (j, i)),
                pl.BlockSpec((2 * HEADS, TR), lambda i, j, o=lofs: (0, i + o)),
                pl.BlockSpec((TL, 2 * HEADS), lambda i, j: (j, 0)),
                pl.BlockSpec((8, 128), lambda i, j: (0, 0)),
            ],
            out_specs=pl.BlockSpec((TR, Dh), lambda i, j, o=lofs: (i + o, 0)),
            out_shape=jax.ShapeDtypeStruct((N, Dh), jnp.float32),
            input_output_aliases={5: 0},
            scratch_shapes=(
                [pltpu.VMEM((HEAD, TR), jnp.float32) for _ in range(HEADS)]
                + [pltpu.VMEM((8, TR), jnp.float32) for _ in range(HEADS)]
                + [pltpu.VMEM((8, TR), jnp.float32) for _ in range(HEADS)]
            ),
            compiler_params=pltpu.CompilerParams(
                dimension_semantics=("parallel", "arbitrary")),
        )(wh, wh, mask, est, edc, pre)

    NB = N // P
    GB = min(8, NB)
    wstack = jnp.stack(Ws)
    astack = jnp.stack(As)
    adstack = jnp.stack(Ad)
    g = pl.pallas_call(
        functools.partial(_gather_kernel, P=P),
        grid=(NB // GB,),
        in_specs=[
            pl.BlockSpec((GB * P, Dh), lambda b: (b, 0)),
            pl.BlockSpec((LAYERS, Dh, Dh), lambda b: (0, 0, 0)),
            pl.BlockSpec((LAYERS, HEADS, HEAD), lambda b: (0, 0, 0)),
            pl.BlockSpec((LAYERS, HEADS, HEAD), lambda b: (0, 0, 0)),
        ],
        out_specs=pl.BlockSpec((GB, 1, Dh), lambda b: (b, 0, 0)),
        out_shape=jax.ShapeDtypeStruct((NB, 1, Dh), jnp.float32),
        compiler_params=pltpu.CompilerParams(
            dimension_semantics=("parallel",)),
    )(h, wstack, astack, adstack)

    g = g.reshape(NB, Dh)
    return g[:B], g[B:]
